# Initial kernel scaffold; baseline (speedup 1.0000x reference)
import jax
import jax.numpy as jnp
from jax.experimental import pallas as pl


def kernel(seqs, table):
    raise NotImplementedError("write your pallas kernel here")


# speedup vs baseline: 4.1674x; 4.1674x over previous
import functools

import jax
import jax.numpy as jnp
from jax import lax
from jax.experimental import pallas as pl
from jax.experimental.pallas import tpu as pltpu
from jax.experimental.pallas import tpu_sc as plsc

_NUM_VOCAB = 100000
_EMBED_DIM = 64
_BATCH = 4096
_HIST = 200
_B = _BATCH * _HIST
_NC, _NS = 2, 16
_NW = _NC * _NS
_B_PER_W = _B // _NW
_GATHER = 128
_CHUNK = 1024
_G_PER_CHUNK = _CHUNK // _GATHER
_N_CHUNKS = _B_PER_W // _CHUNK


@jax.jit
def _sc_gather(idx2d, table):
    mesh = plsc.VectorSubcoreMesh(core_axis_name="c", subcore_axis_name="s")

    @functools.partial(
        pl.kernel,
        out_type=jax.ShapeDtypeStruct((_B, _EMBED_DIM), jnp.float32),
        mesh=mesh,
        scratch_types=[
            pltpu.VMEM((_G_PER_CHUNK, _GATHER), jnp.int32),
            pltpu.VMEM((_CHUNK, _EMBED_DIM), jnp.float32),
            pltpu.SemaphoreType.DMA,
        ],
        compiler_params=pltpu.CompilerParams(use_tc_tiling_on_sc=False),
    )
    def k(idx_hbm, table_hbm, out_hbm, idx_v, rows_v, sem):
        wid = lax.axis_index("s") * _NC + lax.axis_index("c")
        row0 = wid * (_B_PER_W // _GATHER)
        base = wid * _B_PER_W

        @pl.loop(0, _N_CHUNKS)
        def _(ci):
            pltpu.sync_copy(
                idx_hbm.at[pl.ds(row0 + ci * _G_PER_CHUNK, _G_PER_CHUNK)],
                idx_v,
            )
            copies = [
                pltpu.async_copy(
                    table_hbm.at[idx_v.at[j]],
                    rows_v.at[pl.ds(j * _GATHER, _GATHER)],
                    sem,
                )
                for j in range(_G_PER_CHUNK)
            ]
            for c in copies:
                c.wait()
            pltpu.sync_copy(rows_v, out_hbm.at[pl.ds(base + ci * _CHUNK, _CHUNK)])

    return k(idx2d, table)


def kernel(seqs, table):
    idx2d = seqs.astype(jnp.int32).reshape(_B // _GATHER, _GATHER)
    out = _sc_gather(idx2d, table)
    return out.reshape(_BATCH, _HIST, _EMBED_DIM)


# speedup vs baseline: 4.2188x; 1.0123x over previous
import functools

import jax
import jax.numpy as jnp
from jax import lax
from jax.experimental import pallas as pl
from jax.experimental.pallas import tpu as pltpu
from jax.experimental.pallas import tpu_sc as plsc

_NUM_VOCAB = 100000
_EMBED_DIM = 64
_BATCH = 4096
_HIST = 200
_B = _BATCH * _HIST
_NC, _NS = 2, 16
_NW = _NC * _NS
_B_PER_W = _B // _NW
_GATHER = 128
_CHUNK = 640
_G_PER_CHUNK = _CHUNK // _GATHER
_N_CHUNKS = _B_PER_W // _CHUNK


@jax.jit
def _sc_gather(idx2d, table):
    mesh = plsc.VectorSubcoreMesh(core_axis_name="c", subcore_axis_name="s")

    @functools.partial(
        pl.kernel,
        out_type=jax.ShapeDtypeStruct((_B, _EMBED_DIM), jnp.float32),
        mesh=mesh,
        scratch_types=[
            pltpu.VMEM((2 * _G_PER_CHUNK, _GATHER), jnp.int32),
            pltpu.VMEM((2 * _CHUNK, _EMBED_DIM), jnp.float32),
            pltpu.SemaphoreType.DMA,
            pltpu.SemaphoreType.DMA,
            pltpu.SemaphoreType.DMA,
            pltpu.SemaphoreType.DMA,
        ],
        compiler_params=pltpu.CompilerParams(use_tc_tiling_on_sc=False),
    )
    def k(idx_hbm, table_hbm, out_hbm, idx_v, rows_v, g0, g1, s0, s1):
        gsem = (g0, g1)
        ssem = (s0, s1)
        wid = lax.axis_index("s") * _NC + lax.axis_index("c")
        row0 = wid * (_B_PER_W // _GATHER)
        base = wid * _B_PER_W

        def idx_slot(b):
            return idx_v.at[pl.ds(b * _G_PER_CHUNK, _G_PER_CHUNK)]

        def rows_slot(b):
            return rows_v.at[pl.ds(b * _CHUNK, _CHUNK)]

        def fire(ci, b):
            pltpu.sync_copy(
                idx_hbm.at[pl.ds(row0 + ci * _G_PER_CHUNK, _G_PER_CHUNK)],
                idx_slot(b),
            )
            for j in range(_G_PER_CHUNK):
                pltpu.async_copy(
                    table_hbm.at[idx_slot(b).at[j]],
                    rows_slot(b).at[pl.ds(j * _GATHER, _GATHER)],
                    gsem[b],
                )

        def drain(hbm_like, vslot, sem):
            pltpu.make_async_copy(hbm_like, vslot, sem).wait()

        chunk_hbm = out_hbm.at[pl.ds(0, _CHUNK)]
        idx_chunk_hbm = idx_hbm.at[pl.ds(0, _G_PER_CHUNK)]

        fire(0, 0)

        @pl.loop(0, _N_CHUNKS, step=2)
        def _(ci):
            for b in range(2):
                cur = ci + b
                nxt = cur + 1

                @pl.when(nxt < _N_CHUNKS)
                def _():
                    @pl.when(nxt >= 2)
                    def _():
                        drain(chunk_hbm, rows_slot(1 - b), ssem[1 - b])

                    fire(nxt, 1 - b)

                drain(chunk_hbm, rows_slot(b), gsem[b])
                pltpu.async_copy(
                    rows_slot(b),
                    out_hbm.at[pl.ds(base + cur * _CHUNK, _CHUNK)],
                    ssem[b],
                )

        drain(chunk_hbm, rows_slot(0), ssem[0])
        drain(chunk_hbm, rows_slot(1), ssem[1])

    return k(idx2d, table)


def kernel(seqs, table):
    idx2d = seqs.astype(jnp.int32).reshape(_B // _GATHER, _GATHER)
    out = _sc_gather(idx2d, table)
    return out.reshape(_BATCH, _HIST, _EMBED_DIM)


# speedup vs baseline: 5.6068x; 1.3290x over previous
import functools

import jax
import jax.numpy as jnp
from jax import lax
from jax.experimental import pallas as pl
from jax.experimental.pallas import tpu as pltpu
from jax.experimental.pallas import tpu_sc as plsc

_NUM_VOCAB = 100000
_EMBED_DIM = 64
_PAD_DIM = 128
_BATCH = 4096
_HIST = 200
_B = _BATCH * _HIST
_NC, _NS = 2, 16
_NW = _NC * _NS
_B_PER_W = _B // _NW
_GATHER = 128
_IDX_ROWS = _B_PER_W // _GATHER
_G_PER_CHUNK = 2
_CHUNK = _G_PER_CHUNK * _GATHER
_N_CHUNKS = _B_PER_W // _CHUNK


@jax.jit
def _sc_gather(idx2d, table128):
    mesh = plsc.VectorSubcoreMesh(core_axis_name="c", subcore_axis_name="s")

    @functools.partial(
        pl.kernel,
        out_type=jax.ShapeDtypeStruct((_B, _PAD_DIM), jnp.float32),
        mesh=mesh,
        scratch_types=[
            pltpu.VMEM((_IDX_ROWS, _GATHER), jnp.int32),
            pltpu.VMEM((2 * _CHUNK, _PAD_DIM), jnp.float32),
            pltpu.SemaphoreType.DMA,
            pltpu.SemaphoreType.DMA,
            pltpu.SemaphoreType.DMA,
            pltpu.SemaphoreType.DMA,
        ],
    )
    def k(idx_hbm, table_hbm, out_hbm, idx_v, rows_v, g0, g1, s0, s1):
        gsem = (g0, g1)
        ssem = (s0, s1)
        wid = lax.axis_index("s") * _NC + lax.axis_index("c")
        base = wid * _B_PER_W

        pltpu.sync_copy(idx_hbm.at[pl.ds(wid * _IDX_ROWS, _IDX_ROWS)], idx_v)

        def rows_slot(b):
            return rows_v.at[pl.ds(b * _CHUNK, _CHUNK)]

        def fire(ci, b):
            for j in range(_G_PER_CHUNK):
                pltpu.async_copy(
                    table_hbm.at[idx_v.at[ci * _G_PER_CHUNK + j]],
                    rows_slot(b).at[pl.ds(j * _GATHER, _GATHER)],
                    gsem[b],
                )

        def store(ci, b):
            pltpu.async_copy(
                rows_slot(b),
                out_hbm.at[pl.ds(base + ci * _CHUNK, _CHUNK)],
                ssem[b],
            )

        def drain_gather(b):
            pltpu.make_async_copy(
                table_hbm.at[pl.ds(0, _CHUNK)],
                rows_slot(b),
                gsem[b],
            ).wait()

        def drain_store(b):
            pltpu.make_async_copy(
                rows_slot(b),
                out_hbm.at[pl.ds(0, _CHUNK)],
                ssem[b],
            ).wait()

        fire(0, 0)

        @pl.loop(0, _N_CHUNKS, step=2)
        def _(ci):
            for b in range(2):
                cur = ci + b
                nxt = cur + 1

                @pl.when(nxt < _N_CHUNKS)
                def _():
                    @pl.when(nxt >= 2)
                    def _():
                        drain_store(1 - b)

                    fire(nxt, 1 - b)

                drain_gather(b)
                store(cur, b)

        drain_store(0)
        drain_store(1)

    return k(idx2d, table128)


def kernel(seqs, table):
    idx2d = seqs.astype(jnp.int32).reshape(_B // _GATHER, _GATHER)
    table128 = jnp.pad(table, ((0, 0), (0, _PAD_DIM - _EMBED_DIM)))
    out = _sc_gather(idx2d, table128)
    return out[:, :_EMBED_DIM].reshape(_BATCH, _HIST, _EMBED_DIM)


# speedup vs baseline: 7.5426x; 1.3453x over previous
import functools

import jax
import jax.numpy as jnp
from jax import lax
from jax.experimental import pallas as pl
from jax.experimental.pallas import tpu as pltpu
from jax.experimental.pallas import tpu_sc as plsc

_NUM_VOCAB = 100000
_EMBED_DIM = 64
_BATCH = 4096
_HIST = 200
_B = _BATCH * _HIST
_NC, _NS = 2, 16
_NW = _NC * _NS
_B_PER_W = _B // _NW
_GATHER = 128
_IDX_ROWS = _B_PER_W // _GATHER
_G_PER_CHUNK = 4
_CHUNK = _G_PER_CHUNK * _GATHER
_N_CHUNKS = _B_PER_W // _CHUNK


@jax.jit
def _sc_gather(idx2d, table):
    mesh = plsc.VectorSubcoreMesh(core_axis_name="c", subcore_axis_name="s")

    @functools.partial(
        pl.kernel,
        out_type=jax.ShapeDtypeStruct((_B, 2 * _EMBED_DIM), jnp.float32),
        mesh=mesh,
        scratch_types=[
            pltpu.VMEM((_IDX_ROWS, _GATHER), jnp.int32),
            pltpu.VMEM((2 * _CHUNK, _EMBED_DIM), jnp.float32),
            pltpu.SemaphoreType.DMA,
            pltpu.SemaphoreType.DMA,
            pltpu.SemaphoreType.DMA,
            pltpu.SemaphoreType.DMA,
        ],
        compiler_params=pltpu.CompilerParams(use_tc_tiling_on_sc=False),
    )
    def k(idx_hbm, table_hbm, out_hbm, idx_v, rows_v, g0, g1, s0, s1):
        gsem = (g0, g1)
        ssem = (s0, s1)
        wid = lax.axis_index("s") * _NC + lax.axis_index("c")
        base = wid * _B_PER_W

        pltpu.sync_copy(idx_hbm.at[pl.ds(wid * _IDX_ROWS, _IDX_ROWS)], idx_v)

        def rows_slot(b):
            return rows_v.at[pl.ds(b * _CHUNK, _CHUNK)]

        def fire(ci, b):
            for j in range(_G_PER_CHUNK):
                pltpu.async_copy(
                    table_hbm.at[idx_v.at[ci * _G_PER_CHUNK + j]],
                    rows_slot(b).at[pl.ds(j * _GATHER, _GATHER)],
                    gsem[b],
                )

        def store(ci, b):
            pltpu.async_copy(
                rows_slot(b),
                out_hbm.at[pl.ds(base + ci * _CHUNK, _CHUNK), pl.ds(0, _EMBED_DIM)],
                ssem[b],
            )

        def drain_gather(b):
            pltpu.make_async_copy(
                table_hbm.at[pl.ds(0, _CHUNK)],
                rows_slot(b),
                gsem[b],
            ).wait()

        def drain_store(b):
            pltpu.make_async_copy(
                rows_slot(b),
                out_hbm.at[pl.ds(0, _CHUNK), pl.ds(0, _EMBED_DIM)],
                ssem[b],
            ).wait()

        fire(0, 0)

        @pl.loop(0, _N_CHUNKS, step=2)
        def _(ci):
            for b in range(2):
                cur = ci + b
                nxt = cur + 1

                @pl.when(nxt < _N_CHUNKS)
                def _():
                    @pl.when(nxt >= 2)
                    def _():
                        drain_store(1 - b)

                    fire(nxt, 1 - b)

                drain_gather(b)
                store(cur, b)

        drain_store(0)
        drain_store(1)

    return k(idx2d, table)


def kernel(seqs, table):
    idx2d = seqs.astype(jnp.int32).reshape(_B // _GATHER, _GATHER)
    out = _sc_gather(idx2d, table)
    return out[:, :_EMBED_DIM].reshape(_BATCH, _HIST, _EMBED_DIM)


# speedup vs baseline: 7.5496x; 1.0009x over previous
import functools

import jax
import jax.numpy as jnp
from jax import lax
from jax.experimental import pallas as pl
from jax.experimental.pallas import tpu as pltpu
from jax.experimental.pallas import tpu_sc as plsc

_NUM_VOCAB = 100000
_EMBED_DIM = 64
_BATCH = 4096
_HIST = 200
_B = _BATCH * _HIST
_NC, _NS = 2, 16
_NW = _NC * _NS
_B_PER_W = _B // _NW
_GATHER = 128
_G_PER_CHUNK = 4
_CHUNK = _G_PER_CHUNK * _GATHER
_N_CHUNKS = _B_PER_W // _CHUNK


@jax.jit
def _sc_gather(idx2d, table):
    mesh = plsc.VectorSubcoreMesh(core_axis_name="c", subcore_axis_name="s")

    @functools.partial(
        pl.kernel,
        out_type=jax.ShapeDtypeStruct((_B, 2 * _EMBED_DIM), jnp.float32),
        mesh=mesh,
        scratch_types=[
            pltpu.VMEM((_B_PER_W,), jnp.int32),
            pltpu.VMEM((2 * _CHUNK, _EMBED_DIM), jnp.float32),
            pltpu.SemaphoreType.DMA,
            pltpu.SemaphoreType.DMA,
            pltpu.SemaphoreType.DMA,
            pltpu.SemaphoreType.DMA,
        ],
        compiler_params=pltpu.CompilerParams(use_tc_tiling_on_sc=False),
    )
    def k(idx_hbm, table_hbm, out_hbm, idx_v, rows_v, g0, g1, s0, s1):
        gsem = (g0, g1)
        ssem = (s0, s1)
        wid = lax.axis_index("s") * _NC + lax.axis_index("c")
        base = wid * _B_PER_W

        pltpu.sync_copy(idx_hbm.at[pl.ds(wid * _B_PER_W, _B_PER_W)], idx_v)

        def rows_slot(b):
            return rows_v.at[pl.ds(b * _CHUNK, _CHUNK)]

        def fire(ci, b):
            for j in range(_G_PER_CHUNK):
                pltpu.async_copy(
                    table_hbm.at[idx_v.at[pl.ds((ci * _G_PER_CHUNK + j) * _GATHER, _GATHER)]],
                    rows_slot(b).at[pl.ds(j * _GATHER, _GATHER)],
                    gsem[b],
                )

        def store(ci, b):
            pltpu.async_copy(
                rows_slot(b),
                out_hbm.at[pl.ds(base + ci * _CHUNK, _CHUNK), pl.ds(0, _EMBED_DIM)],
                ssem[b],
            )

        def drain_gather(b):
            pltpu.make_async_copy(
                table_hbm.at[pl.ds(0, _CHUNK)],
                rows_slot(b),
                gsem[b],
            ).wait()

        def drain_store(b):
            pltpu.make_async_copy(
                rows_slot(b),
                out_hbm.at[pl.ds(0, _CHUNK), pl.ds(0, _EMBED_DIM)],
                ssem[b],
            ).wait()

        fire(0, 0)

        @pl.loop(0, _N_CHUNKS, step=2)
        def _(ci):
            for b in range(2):
                cur = ci + b
                nxt = cur + 1

                @pl.when(nxt < _N_CHUNKS)
                def _():
                    @pl.when(nxt >= 2)
                    def _():
                        drain_store(1 - b)

                    fire(nxt, 1 - b)

                drain_gather(b)
                store(cur, b)

        drain_store(0)
        drain_store(1)

    return k(idx2d, table)


def kernel(seqs, table):
    idx1d = seqs.astype(jnp.int32).reshape(_B)
    out = _sc_gather(idx1d, table)
    return out[:, :_EMBED_DIM].reshape(_BATCH, _HIST, _EMBED_DIM)
